# Initial kernel scaffold; baseline (speedup 1.0000x reference)
import jax
import jax.numpy as jnp
from jax.experimental import pallas as pl


def kernel(x, edge_index, W0l, b0l, W0r, W1l, b1l, W1r):
    raise NotImplementedError("write your pallas kernel here")


# speedup vs baseline: 7.3382x; 7.3382x over previous
import functools

import jax
import jax.numpy as jnp
from jax import lax
from jax.experimental import pallas as pl
from jax.experimental.pallas import tpu as pltpu
from jax.experimental.pallas import tpu_sc as plsc

N = 10000
E = 320000
D = 128
NC = 2
NS = 16
NW = NC * NS
BPW = 256
B = NW * BPW

mesh = plsc.VectorSubcoreMesh(core_axis_name="c", subcore_axis_name="s")

NP = 10112
RPT = NP // NS


K = 128
CB = 8
CH = 80
EPT = CH * K
EPAD = NW * EPT


K = 128
CB = 8
CH = 80
EPT = CH * K
EPAD = NW * EPT
CW = 16


def _make_sc_agg(with_counts):
    out_type = [jax.ShapeDtypeStruct((NC, NP, D), jnp.float32)]
    scratch = [
        pltpu.VMEM((CB, K), jnp.int32),
        pltpu.VMEM((CB, K), jnp.int32),
        pltpu.VMEM((K, D), jnp.float32),
        pltpu.VMEM_SHARED((NP, D), jnp.float32),
    ]
    if with_counts:
        out_type.append(jax.ShapeDtypeStruct((NC * NP,), jnp.float32))
        scratch += [
            pltpu.VMEM((K,), jnp.float32),
            pltpu.VMEM((RPT,), jnp.float32),
            pltpu.VMEM_SHARED((NP,), jnp.float32),
        ]

    def body(table_hbm, sidx_hbm, didx_hbm, zrow_hbm, *rest):
        if with_counts:
            (zcnt_hbm, ones_hbm, out_hbm, cnts_out,
             sidx_v, didx_v, rows_v, acc_sh, ones_v, cbuf_v, cnt_sh) = rest
        else:
            out_hbm, sidx_v, didx_v, rows_v, acc_sh = rest
        c = lax.axis_index("c")
        s = lax.axis_index("s")
        row0 = s * RPT
        pieces = [(o, min(K, RPT - o)) for o in range(0, RPT, K)]
        pltpu.sync_copy(zrow_hbm, rows_v)
        for o, sz in pieces:
            pltpu.sync_copy(rows_v.at[pl.ds(0, sz)],
                            acc_sh.at[pl.ds(row0 + o, sz)])
        if with_counts:
            pltpu.sync_copy(zcnt_hbm.at[pl.ds(0, RPT)], cbuf_v)
            pltpu.sync_copy(cbuf_v, cnt_sh.at[pl.ds(row0, RPT)])
            pltpu.sync_copy(ones_hbm, ones_v)
        plsc.subcore_barrier()

        def blk(bi, carry):
            pltpu.sync_copy(sidx_hbm.at[c, s, pl.ds(bi * CB, CB)], sidx_v)
            pltpu.sync_copy(didx_hbm.at[c, s, pl.ds(bi * CB, CB)], didx_v)
            for j in range(CB):
                pltpu.sync_copy(table_hbm.at[sidx_v.at[j]], rows_v)
                pltpu.sync_copy(rows_v, acc_sh.at[didx_v.at[j]], add=True)
                if with_counts:
                    pltpu.sync_copy(ones_v, cnt_sh.at[didx_v.at[j]],
                                    add=True)
            return carry

        lax.fori_loop(0, CH // CB, blk, 0)
        plsc.subcore_barrier()
        for o, sz in pieces:
            pltpu.sync_copy(acc_sh.at[pl.ds(row0 + o, sz)],
                            rows_v.at[pl.ds(0, sz)])
            pltpu.sync_copy(rows_v.at[pl.ds(0, sz)],
                            out_hbm.at[c, pl.ds(row0 + o, sz)])
        if with_counts:
            pltpu.sync_copy(cnt_sh.at[pl.ds(row0, RPT)], cbuf_v)
            pltpu.sync_copy(cbuf_v, cnts_out.at[pl.ds(c * NP + row0, RPT)])

    return pl.kernel(body, out_type=tuple(out_type) if with_counts
                     else out_type[0],
                     mesh=mesh, scratch_types=scratch)


_sc_agg_counts = _make_sc_agg(True)
_sc_agg = _make_sc_agg(False)


BR = 1000
GR = N // BR

_ROWS = pl.BlockSpec((BR, D), lambda i: (i, 0))
_CNTR = pl.BlockSpec((BR, 1), lambda i: (i, 0))
_WMAT = pl.BlockSpec((D, D), lambda i: (0, 0))
_BIAS = pl.BlockSpec((1, D), lambda i: (0, 0))


def _mm_body(x_ref, w_ref, o_ref):
    o_ref[...] = jnp.dot(x_ref[...], w_ref[...],
                         preferred_element_type=jnp.float32)


_premul = pl.pallas_call(
    _mm_body, grid=(GR,),
    in_specs=[_ROWS, _WMAT], out_specs=_ROWS,
    out_shape=jax.ShapeDtypeStruct((N, D), jnp.float32))


def _mid_body(x_ref, sa_ref, sb_ref, ca_ref, cb_ref, b_ref, wr_ref, wl_ref,
              h_ref, y1_ref):
    cnt = jnp.maximum(ca_ref[...] + cb_ref[...], 1.0)
    mean = (sa_ref[...] + sb_ref[...]) / cnt
    h = mean + b_ref[...] + jnp.dot(x_ref[...], wr_ref[...],
                                    preferred_element_type=jnp.float32)
    h = jnp.maximum(h, 0.0)
    h_ref[...] = h
    y1_ref[...] = jnp.dot(h, wl_ref[...], preferred_element_type=jnp.float32)


_mid = pl.pallas_call(
    _mid_body, grid=(GR,),
    in_specs=[_ROWS, _ROWS, _ROWS, _CNTR, _CNTR, _BIAS, _WMAT, _WMAT],
    out_specs=(_ROWS, _ROWS),
    out_shape=(jax.ShapeDtypeStruct((N, D), jnp.float32),
               jax.ShapeDtypeStruct((N, D), jnp.float32)))


def _post_body(h_ref, sa_ref, sb_ref, ca_ref, cb_ref, b_ref, wr_ref, o_ref):
    cnt = jnp.maximum(ca_ref[...] + cb_ref[...], 1.0)
    o = (sa_ref[...] + sb_ref[...]) / cnt + b_ref[...] + jnp.dot(
        h_ref[...], wr_ref[...], preferred_element_type=jnp.float32)
    norm = jnp.sqrt(jnp.sum(o * o, axis=1, keepdims=True))
    o_ref[...] = o / jnp.maximum(norm, 1e-12)


_post = pl.pallas_call(
    _post_body, grid=(GR,),
    in_specs=[_ROWS, _ROWS, _ROWS, _CNTR, _CNTR, _BIAS, _WMAT],
    out_specs=_ROWS,
    out_shape=jax.ShapeDtypeStruct((N, D), jnp.float32))


def kernel(x, edge_index, W0l, b0l, W0r, W1l, b1l, W1r):
    src = edge_index[0].astype(jnp.int32)
    dst = edge_index[1].astype(jnp.int32)
    pad_i = jnp.arange(EPAD - E, dtype=jnp.int32)
    srcp = jnp.concatenate([src, (pad_i * 997) % N]).reshape(NC, NS, CH, K)
    dstp = jnp.concatenate([dst, N + (pad_i % (NP - N))]).reshape(
        NC, NS, CH, K)
    zrow = jnp.zeros((K, D), jnp.float32)
    zcnt = jnp.zeros((NP,), jnp.float32)
    ones = jnp.ones((K,), jnp.float32)

    y0 = _premul(x, W0l.T)
    s0, c0 = _sc_agg_counts(y0, srcp, dstp, zrow, zcnt, ones)
    c0 = c0.reshape(NC, NP)
    ca = c0[0, :N, None]
    cb = c0[1, :N, None]
    h, y1 = _mid(x, s0[0, :N], s0[1, :N], ca, cb,
                 b0l.reshape(1, D), W0r.T, W1l.T)
    s1 = _sc_agg(y1, srcp, dstp, zrow)
    return _post(h, s1[0, :N], s1[1, :N], ca, cb,
                 b1l.reshape(1, D), W1r.T)


# speedup vs baseline: 11.1212x; 1.5155x over previous
import functools

import jax
import jax.numpy as jnp
from jax import lax
from jax.experimental import pallas as pl
from jax.experimental.pallas import tpu as pltpu
from jax.experimental.pallas import tpu_sc as plsc

N = 10000
E = 320000
D = 128
NC = 2
NS = 16
NW = NC * NS
BPW = 256
B = NW * BPW

mesh = plsc.VectorSubcoreMesh(core_axis_name="c", subcore_axis_name="s")

NP = 10112
RPT = NP // NS


K = 128
CB = 8
CH = 80
EPT = CH * K
EPAD = NW * EPT


K = 128
CB = 8
CH = 80
EPT = CH * K
EPAD = NW * EPT
CW = 16


NB = CH // CB


def _make_sc_agg(with_counts):
    out_type = [jax.ShapeDtypeStruct((NC, NP, D), jnp.float32)]
    scratch = [
        pltpu.VMEM((2, CB, K), jnp.int32),
        pltpu.VMEM((2, CB, K), jnp.int32),
        pltpu.VMEM((2, K, D), jnp.float32),
        pltpu.VMEM_SHARED((NP, D), jnp.float32),
        pltpu.SemaphoreType.DMA,
        pltpu.SemaphoreType.DMA,
        pltpu.SemaphoreType.DMA,
        pltpu.SemaphoreType.DMA,
        pltpu.SemaphoreType.DMA,
        pltpu.SemaphoreType.DMA,
        pltpu.SemaphoreType.DMA,
        pltpu.SemaphoreType.DMA,
    ]
    if with_counts:
        out_type.append(jax.ShapeDtypeStruct((NC * NP,), jnp.float32))
        scratch += [
            pltpu.VMEM((K,), jnp.float32),
            pltpu.VMEM((RPT,), jnp.float32),
            pltpu.VMEM_SHARED((NP,), jnp.float32),
            pltpu.SemaphoreType.DMA,
            pltpu.SemaphoreType.DMA,
        ]

    def body(table_hbm, sidx_hbm, didx_hbm, zrow_hbm, *rest):
        if with_counts:
            (zcnt_hbm, ones_hbm, out_hbm, cnts_out, sidx_v, didx_v, rows_v,
             acc_sh, gs0, gs1, ss0, ss1, is0, is1, id0, id1,
             ones_v, cbuf_v, cnt_sh, cs0, cs1) = rest
            csem = [cs0, cs1]
        else:
            (out_hbm, sidx_v, didx_v, rows_v, acc_sh,
             gs0, gs1, ss0, ss1, is0, is1, id0, id1) = rest
        gsem = [gs0, gs1]
        ssem = [ss0, ss1]
        isem_s = [is0, is1]
        isem_d = [id0, id1]
        c = lax.axis_index("c")
        s = lax.axis_index("s")
        row0 = s * RPT
        pieces = [(o, min(K, RPT - o)) for o in range(0, RPT, K)]

        pend_i = {}
        pend_g = {}
        pend_s = {}
        pend_c = {}

        def issue_idx(bi):
            sl = bi % 2
            pend_i[sl] = (
                pltpu.async_copy(sidx_hbm.at[c, s, pl.ds(bi * CB, CB)],
                                 sidx_v.at[sl], isem_s[sl]),
                pltpu.async_copy(didx_hbm.at[c, s, pl.ds(bi * CB, CB)],
                                 didx_v.at[sl], isem_d[sl]))

        def issue_gather(t):
            bi, j = divmod(t, CB)
            sl = bi % 2
            p = t % 2
            if j == 0 and sl in pend_i:
                for d in pend_i.pop(sl):
                    d.wait()
            pend_g[p] = pltpu.async_copy(
                table_hbm.at[sidx_v.at[sl, j]], rows_v.at[p], gsem[p])

        issue_idx(0)
        if NB > 1:
            issue_idx(1)

        pltpu.sync_copy(zrow_hbm, rows_v.at[0])
        for o, sz in pieces:
            pltpu.sync_copy(rows_v.at[0, pl.ds(0, sz)],
                            acc_sh.at[pl.ds(row0 + o, sz)])
        if with_counts:
            pltpu.sync_copy(zcnt_hbm.at[pl.ds(0, RPT)], cbuf_v)
            pltpu.sync_copy(cbuf_v, cnt_sh.at[pl.ds(row0, RPT)])
            pltpu.sync_copy(ones_hbm, ones_v)
        plsc.subcore_barrier()

        issue_gather(0)
        for t in range(CH):
            bi, j = divmod(t, CB)
            p = t % 2
            if t + 1 < CH:
                if (1 - p) in pend_s:
                    pend_s.pop(1 - p).wait()
                issue_gather(t + 1)
            pend_g.pop(p).wait()
            if p in pend_s:
                pend_s.pop(p).wait()
            sl = bi % 2
            pend_s[p] = pltpu.async_copy(
                rows_v.at[p], acc_sh.at[didx_v.at[sl, j]], ssem[p], add=True)
            if with_counts:
                if p in pend_c:
                    pend_c.pop(p).wait()
                pend_c[p] = pltpu.async_copy(
                    ones_v, cnt_sh.at[didx_v.at[sl, j]], csem[p], add=True)
            if j == 1 and 2 <= bi + 1 < NB:
                issue_idx(bi + 1)
        for d in list(pend_s.values()) + list(pend_c.values()):
            d.wait()
        plsc.subcore_barrier()
        for o, sz in pieces:
            pltpu.sync_copy(acc_sh.at[pl.ds(row0 + o, sz)],
                            rows_v.at[0, pl.ds(0, sz)])
            pltpu.sync_copy(rows_v.at[0, pl.ds(0, sz)],
                            out_hbm.at[c, pl.ds(row0 + o, sz)])
        if with_counts:
            pltpu.sync_copy(cnt_sh.at[pl.ds(row0, RPT)], cbuf_v)
            pltpu.sync_copy(cbuf_v, cnts_out.at[pl.ds(c * NP + row0, RPT)])

    return pl.kernel(body, out_type=tuple(out_type) if with_counts
                     else out_type[0],
                     mesh=mesh, scratch_types=scratch)


_sc_agg_counts = _make_sc_agg(True)
_sc_agg = _make_sc_agg(False)


BR = 1000
GR = N // BR

_ROWS = pl.BlockSpec((BR, D), lambda i: (i, 0))
_CNTR = pl.BlockSpec((BR, 1), lambda i: (i, 0))
_WMAT = pl.BlockSpec((D, D), lambda i: (0, 0))
_BIAS = pl.BlockSpec((1, D), lambda i: (0, 0))


def _mm_body(x_ref, w_ref, o_ref):
    o_ref[...] = jnp.dot(x_ref[...], w_ref[...],
                         preferred_element_type=jnp.float32)


_premul = pl.pallas_call(
    _mm_body, grid=(GR,),
    in_specs=[_ROWS, _WMAT], out_specs=_ROWS,
    out_shape=jax.ShapeDtypeStruct((N, D), jnp.float32))


def _mid_body(x_ref, sa_ref, sb_ref, ca_ref, cb_ref, b_ref, wr_ref, wl_ref,
              h_ref, y1_ref):
    cnt = jnp.maximum(ca_ref[...] + cb_ref[...], 1.0)
    mean = (sa_ref[...] + sb_ref[...]) / cnt
    h = mean + b_ref[...] + jnp.dot(x_ref[...], wr_ref[...],
                                    preferred_element_type=jnp.float32)
    h = jnp.maximum(h, 0.0)
    h_ref[...] = h
    y1_ref[...] = jnp.dot(h, wl_ref[...], preferred_element_type=jnp.float32)


_mid = pl.pallas_call(
    _mid_body, grid=(GR,),
    in_specs=[_ROWS, _ROWS, _ROWS, _CNTR, _CNTR, _BIAS, _WMAT, _WMAT],
    out_specs=(_ROWS, _ROWS),
    out_shape=(jax.ShapeDtypeStruct((N, D), jnp.float32),
               jax.ShapeDtypeStruct((N, D), jnp.float32)))


def _post_body(h_ref, sa_ref, sb_ref, ca_ref, cb_ref, b_ref, wr_ref, o_ref):
    cnt = jnp.maximum(ca_ref[...] + cb_ref[...], 1.0)
    o = (sa_ref[...] + sb_ref[...]) / cnt + b_ref[...] + jnp.dot(
        h_ref[...], wr_ref[...], preferred_element_type=jnp.float32)
    norm = jnp.sqrt(jnp.sum(o * o, axis=1, keepdims=True))
    o_ref[...] = o / jnp.maximum(norm, 1e-12)


_post = pl.pallas_call(
    _post_body, grid=(GR,),
    in_specs=[_ROWS, _ROWS, _ROWS, _CNTR, _CNTR, _BIAS, _WMAT],
    out_specs=_ROWS,
    out_shape=jax.ShapeDtypeStruct((N, D), jnp.float32))


def kernel(x, edge_index, W0l, b0l, W0r, W1l, b1l, W1r):
    src = edge_index[0].astype(jnp.int32)
    dst = edge_index[1].astype(jnp.int32)
    pad_i = jnp.arange(EPAD - E, dtype=jnp.int32)
    srcp = jnp.concatenate([src, (pad_i * 997) % N]).reshape(NC, NS, CH, K)
    dstp = jnp.concatenate([dst, N + (pad_i % (NP - N))]).reshape(
        NC, NS, CH, K)
    zrow = jnp.zeros((K, D), jnp.float32)
    zcnt = jnp.zeros((NP,), jnp.float32)
    ones = jnp.ones((K,), jnp.float32)

    y0 = _premul(x, W0l.T)
    s0, c0 = _sc_agg_counts(y0, srcp, dstp, zrow, zcnt, ones)
    c0 = c0.reshape(NC, NP)
    ca = c0[0, :N, None]
    cb = c0[1, :N, None]
    h, y1 = _mid(x, s0[0, :N], s0[1, :N], ca, cb,
                 b0l.reshape(1, D), W0r.T, W1l.T)
    s1 = _sc_agg(y1, srcp, dstp, zrow)
    return _post(h, s1[0, :N], s1[1, :N], ca, cb,
                 b1l.reshape(1, D), W1r.T)
